# Initial kernel scaffold; baseline (speedup 1.0000x reference)
import jax
import jax.numpy as jnp
from jax.experimental import pallas as pl


def kernel(x, edge_index, W1, b1, g1, be1, m1, v1, W2, b2, g2, be2, m2, v2, W3, b3, g3, be3, m3, v3, W4, b4):
    raise NotImplementedError("write your pallas kernel here")


# speedup vs baseline: 10.3521x; 10.3521x over previous
import functools

import jax
import jax.numpy as jnp
from jax import lax
from jax.experimental import pallas as pl
from jax.experimental.pallas import tpu as pltpu
from jax.experimental.pallas import tpu_sc as plsc

_N = 10000
_D = 256
_H = 256
_NP = 10240
_EP = 163840
_ROWS = _EP // 128
_RPT = _ROWS // 16
_RPW = _ROWS // 32
_ZR = _NP // 16
_BN = 1000
_GRID = _N // _BN

_mesh = plsc.VectorSubcoreMesh(core_axis_name="c", subcore_axis_name="s")


def _seg_sum_wide(u0, u1, src2d, dst2d, zeros):
    @functools.partial(
        pl.kernel,
        out_type=(
            jax.ShapeDtypeStruct((_NP, 128), jnp.float32),
            jax.ShapeDtypeStruct((_NP, 128), jnp.float32),
        ),
        mesh=_mesh,
        scratch_types=[
            pltpu.VMEM((_RPT, 128), jnp.int32),
            pltpu.VMEM((_RPT, 128), jnp.int32),
            pltpu.VMEM((128, 128), jnp.float32),
            pltpu.VMEM_SHARED((_NP, 128), jnp.float32),
            pltpu.SemaphoreType.DMA,
        ],
    )
    def k(u0h, u1h, srch, dsth, zh, out0, out1, src_v, dst_v, rows_v,
          agg_sh, sem):
        core = lax.axis_index("c")
        tid = lax.axis_index("s")

        def run(table, out):
            pltpu.sync_copy(srch.at[pl.ds(tid * _RPT, _RPT)], src_v)
            pltpu.sync_copy(dsth.at[pl.ds(tid * _RPT, _RPT)], dst_v)
            for z in range(_ZR // 128):
                pltpu.sync_copy(
                    zh, agg_sh.at[pl.ds(tid * _ZR + z * 128, 128)])
            plsc.subcore_barrier()

            def body(j, carry):
                pltpu.async_copy(table.at[src_v.at[j]], rows_v, sem).wait()
                pltpu.sync_copy(rows_v, agg_sh.at[dst_v.at[j]], add=True)
                return carry

            lax.fori_loop(0, _RPT, body, 0)
            plsc.subcore_barrier()
            pltpu.sync_copy(agg_sh.at[pl.ds(tid * _ZR, _ZR)],
                            out.at[pl.ds(tid * _ZR, _ZR)])

        @pl.when(core == 0)
        def _():
            run(u0h, out0)

        @pl.when(core == 1)
        def _():
            run(u1h, out1)

    return k(u0, u1, src2d, dst2d, zeros)


def _seg_sum_edge(table, src2d, dst2d, zeros, gather=True):
    @functools.partial(
        pl.kernel,
        out_type=(
            jax.ShapeDtypeStruct((_NP, 128), jnp.float32),
            jax.ShapeDtypeStruct((_NP, 128), jnp.float32),
        ),
        mesh=_mesh,
        scratch_types=[
            pltpu.VMEM((_RPW, 128), jnp.int32),
            pltpu.VMEM((_RPW, 128), jnp.int32),
            pltpu.VMEM((128, 128), jnp.float32),
            pltpu.VMEM_SHARED((_NP, 128), jnp.float32),
            pltpu.SemaphoreType.DMA,
        ],
    )
    def k(th, srch, dsth, zh, out0, out1, src_v, dst_v, rows_v, agg_sh, sem):
        core = lax.axis_index("c")
        tid = lax.axis_index("s")

        def run(plane, out):
            wid = plane * 16 + tid
            if gather:
                pltpu.sync_copy(srch.at[pl.ds(wid * _RPW, _RPW)], src_v)
            else:
                pltpu.sync_copy(th.at[pl.ds(0, 128)], rows_v)
            pltpu.sync_copy(dsth.at[pl.ds(wid * _RPW, _RPW)], dst_v)
            for z in range(_ZR // 128):
                pltpu.sync_copy(
                    zh, agg_sh.at[pl.ds(tid * _ZR + z * 128, 128)])
            plsc.subcore_barrier()

            def body(j, carry):
                if gather:
                    pltpu.async_copy(th.at[src_v.at[j]], rows_v, sem).wait()
                pltpu.sync_copy(rows_v, agg_sh.at[dst_v.at[j]], add=True)
                return carry

            lax.fori_loop(0, _RPW, body, 0)
            plsc.subcore_barrier()
            pltpu.sync_copy(agg_sh.at[pl.ds(tid * _ZR, _ZR)],
                            out.at[pl.ds(tid * _ZR, _ZR)])

        @pl.when(core == 0)
        def _():
            run(0, out0)

        @pl.when(core == 1)
        def _():
            run(1, out1)

    return k(table, src2d, dst2d, zeros)


def _tc_first(x, W1, dp0, dp1):
    def body(x_ref, w_ref, d0_ref, d1_ref, u0_ref, u1_ref, di_ref):
        deg = d0_ref[:, 0:1] + d1_ref[:, 0:1] + 1.0
        dinv = 1.0 / jnp.sqrt(deg)
        u = jnp.dot(x_ref[...], w_ref[...],
                    preferred_element_type=jnp.float32)
        up = dinv * u
        u0_ref[...] = up[:, :128]
        u1_ref[...] = up[:, 128:]
        di_ref[...] = dinv

    return pl.pallas_call(
        body,
        grid=(_GRID,),
        in_specs=[
            pl.BlockSpec((_BN, _D), lambda i: (i, 0)),
            pl.BlockSpec((_D, _H), lambda i: (0, 0)),
            pl.BlockSpec((_BN, 128), lambda i: (i, 0)),
            pl.BlockSpec((_BN, 128), lambda i: (i, 0)),
        ],
        out_specs=[
            pl.BlockSpec((_BN, 128), lambda i: (i, 0)),
            pl.BlockSpec((_BN, 128), lambda i: (i, 0)),
            pl.BlockSpec((_BN, 1), lambda i: (i, 0)),
        ],
        out_shape=[
            jax.ShapeDtypeStruct((_N, 128), jnp.float32),
            jax.ShapeDtypeStruct((_N, 128), jnp.float32),
            jax.ShapeDtypeStruct((_N, 1), jnp.float32),
        ],
    )(x, W1, dp0, dp1)


def _tc_mid(agg0, agg1, u0, u1, dinv, W, gamma, delta, last):
    def body(a0_ref, a1_ref, u0_ref, u1_ref, di_ref, w_ref, g_ref, dl_ref,
             o1_ref, o2_ref):
        dinv_b = di_ref[...]
        s = jnp.concatenate(
            [a0_ref[...] + u0_ref[...], a1_ref[...] + u1_ref[...]], axis=1)
        h = jnp.maximum(g_ref[...] * (dinv_b * s) + dl_ref[...], 0.0)
        u = jnp.dot(h, w_ref[...], preferred_element_type=jnp.float32)
        up = dinv_b * u
        if last:
            o1_ref[...] = up
            o2_ref[...] = jnp.broadcast_to(up, (up.shape[0], 128))
        else:
            o1_ref[...] = up[:, :128]
            o2_ref[...] = up[:, 128:]

    wcols = 1 if last else _H
    out_specs = (
        [pl.BlockSpec((_BN, 1), lambda i: (i, 0)),
         pl.BlockSpec((_BN, 128), lambda i: (i, 0))]
        if last else
        [pl.BlockSpec((_BN, 128), lambda i: (i, 0)),
         pl.BlockSpec((_BN, 128), lambda i: (i, 0))]
    )
    out_shape = (
        [jax.ShapeDtypeStruct((_N, 1), jnp.float32),
         jax.ShapeDtypeStruct((_N, 128), jnp.float32)]
        if last else
        [jax.ShapeDtypeStruct((_N, 128), jnp.float32),
         jax.ShapeDtypeStruct((_N, 128), jnp.float32)]
    )
    return pl.pallas_call(
        body,
        grid=(_GRID,),
        in_specs=[
            pl.BlockSpec((_BN, 128), lambda i: (i, 0)),
            pl.BlockSpec((_BN, 128), lambda i: (i, 0)),
            pl.BlockSpec((_BN, 128), lambda i: (i, 0)),
            pl.BlockSpec((_BN, 128), lambda i: (i, 0)),
            pl.BlockSpec((_BN, 1), lambda i: (i, 0)),
            pl.BlockSpec((_H, wcols), lambda i: (0, 0)),
            pl.BlockSpec((1, _H), lambda i: (0, 0)),
            pl.BlockSpec((1, _H), lambda i: (0, 0)),
        ],
        out_specs=out_specs,
        out_shape=out_shape,
    )(agg0, agg1, u0, u1, dinv, W, gamma, delta)


def _tc_final(p0, p1, u4p, dinv, b4):
    def body(p0_ref, p1_ref, u_ref, di_ref, b_ref, o_ref):
        a = p0_ref[:, 0:1] + p1_ref[:, 0:1]
        o_ref[...] = di_ref[...] * (a + u_ref[...]) + b_ref[0, 0]

    return pl.pallas_call(
        body,
        grid=(_GRID,),
        in_specs=[
            pl.BlockSpec((_BN, 128), lambda i: (i, 0)),
            pl.BlockSpec((_BN, 128), lambda i: (i, 0)),
            pl.BlockSpec((_BN, 1), lambda i: (i, 0)),
            pl.BlockSpec((_BN, 1), lambda i: (i, 0)),
            pl.BlockSpec((1, 1), lambda i: (0, 0)),
        ],
        out_specs=pl.BlockSpec((_BN, 1), lambda i: (i, 0)),
        out_shape=jax.ShapeDtypeStruct((_N, 1), jnp.float32),
    )(p0, p1, u4p, dinv, b4)


def kernel(x, edge_index, W1, b1, g1, be1, m1, v1, W2, b2, g2, be2, m2, v2,
           W3, b3, g3, be3, m3, v3, W4, b4):
    E = edge_index.shape[1]
    pad = _EP - E
    ar = jnp.arange(pad, dtype=jnp.int32)
    src = jnp.concatenate([edge_index[0], (ar * 97) % _N])
    dst = jnp.concatenate([edge_index[1], _N + (ar % (_NP - _N))])
    src2d = src.reshape(_ROWS, 128)
    dst2d = dst.reshape(_ROWS, 128)
    zeros = jnp.zeros((128, 128), jnp.float32)
    ones = jnp.ones((128, 128), jnp.float32)

    eps = 1e-5
    g1a = g1 / jnp.sqrt(v1 + eps)
    g2a = g2 / jnp.sqrt(v2 + eps)
    g3a = g3 / jnp.sqrt(v3 + eps)
    d1 = (g1a * (b1 - m1) + be1).reshape(1, _H)
    d2 = (g2a * (b2 - m2) + be2).reshape(1, _H)
    d3 = (g3a * (b3 - m3) + be3).reshape(1, _H)
    g1a = g1a.reshape(1, _H)
    g2a = g2a.reshape(1, _H)
    g3a = g3a.reshape(1, _H)

    dp0, dp1 = _seg_sum_edge(ones, src2d, dst2d, zeros, gather=False)
    u0, u1, dinv = _tc_first(x, W1, dp0[:_N], dp1[:_N])

    a0, a1 = _seg_sum_wide(u0, u1, src2d, dst2d, zeros)
    u0, u1 = _tc_mid(a0[:_N], a1[:_N], u0, u1, dinv, W2, g1a, d1, last=False)

    a0, a1 = _seg_sum_wide(u0, u1, src2d, dst2d, zeros)
    u0, u1 = _tc_mid(a0[:_N], a1[:_N], u0, u1, dinv, W3, g2a, d2, last=False)

    a0, a1 = _seg_sum_wide(u0, u1, src2d, dst2d, zeros)
    u4p, u4p128 = _tc_mid(a0[:_N], a1[:_N], u0, u1, dinv, W4.reshape(_H, 1),
                          g3a, d3, last=True)

    p0, p1 = _seg_sum_edge(u4p128, src2d, dst2d, zeros)
    out = _tc_final(p0[:_N], p1[:_N], u4p, dinv, b4.reshape(1, 1))
    return out.reshape(-1)


# speedup vs baseline: 14.3071x; 1.3820x over previous
import functools

import jax
import jax.numpy as jnp
from jax import lax
from jax.experimental import pallas as pl
from jax.experimental.pallas import tpu as pltpu
from jax.experimental.pallas import tpu_sc as plsc

_N = 10000
_D = 256
_H = 256
_NP = 10240
_EP = 163840
_ROWS = _EP // 128
_RPT = _ROWS // 16
_RPW = _ROWS // 32
_ZR = _NP // 16
_BN = 1000
_GRID = _N // _BN

_mesh = plsc.VectorSubcoreMesh(core_axis_name="c", subcore_axis_name="s")


def _pipe(table, srch, dsth, src_v, dst_v, rows0, rows1, agg_sh, sem0, sem1,
          idx_base, nchunks):
    pltpu.sync_copy(srch.at[pl.ds(idx_base, nchunks)], src_v)
    pltpu.sync_copy(dsth.at[pl.ds(idx_base, nchunks)], dst_v)

    def start(j, buf, sem):
        pltpu.async_copy(table.at[src_v.at[j]], buf, sem)

    def wait(buf, sem):
        pltpu.make_async_copy(table.at[src_v.at[0]], buf, sem).wait()

    start(0, rows0, sem0)
    start(1, rows1, sem1)

    def body(j2, carry):
        b = 2 * j2
        wait(rows0, sem0)
        pltpu.sync_copy(rows0, agg_sh.at[dst_v.at[b]], add=True)
        start(b + 2, rows0, sem0)
        wait(rows1, sem1)
        pltpu.sync_copy(rows1, agg_sh.at[dst_v.at[b + 1]], add=True)
        start(b + 3, rows1, sem1)
        return carry

    lax.fori_loop(0, nchunks // 2 - 1, body, 0)
    wait(rows0, sem0)
    pltpu.sync_copy(rows0, agg_sh.at[dst_v.at[nchunks - 2]], add=True)
    wait(rows1, sem1)
    pltpu.sync_copy(rows1, agg_sh.at[dst_v.at[nchunks - 1]], add=True)


def _seg_sum_wide(u0, u1, src2d, dst2d, zeros):
    @functools.partial(
        pl.kernel,
        out_type=(
            jax.ShapeDtypeStruct((_NP, 128), jnp.float32),
            jax.ShapeDtypeStruct((_NP, 128), jnp.float32),
        ),
        mesh=_mesh,
        scratch_types=[
            pltpu.VMEM((_RPT // 2, 128), jnp.int32),
            pltpu.VMEM((_RPT // 2, 128), jnp.int32),
            pltpu.VMEM((128, 128), jnp.float32),
            pltpu.VMEM((128, 128), jnp.float32),
            pltpu.VMEM_SHARED((_NP, 128), jnp.float32),
            pltpu.SemaphoreType.DMA,
            pltpu.SemaphoreType.DMA,
        ],
    )
    def k(u0h, u1h, srch, dsth, zh, out0, out1, src_v, dst_v, rows0, rows1,
          agg_sh, sem0, sem1):
        core = lax.axis_index("c")
        tid = lax.axis_index("s")

        def run(table, out):
            for z in range(_ZR // 128):
                pltpu.sync_copy(
                    zh, agg_sh.at[pl.ds(tid * _ZR + z * 128, 128)])
            plsc.subcore_barrier()
            for phase in range(2):
                _pipe(table, srch, dsth, src_v, dst_v, rows0, rows1,
                      agg_sh, sem0, sem1,
                      tid * _RPT + phase * (_RPT // 2), _RPT // 2)
            plsc.subcore_barrier()
            pltpu.sync_copy(agg_sh.at[pl.ds(tid * _ZR, _ZR)],
                            out.at[pl.ds(tid * _ZR, _ZR)])

        @pl.when(core == 0)
        def _():
            run(u0h, out0)

        @pl.when(core == 1)
        def _():
            run(u1h, out1)

    return k(u0, u1, src2d, dst2d, zeros)


def _seg_sum_edge(table, src2d, dst2d, zeros, gather=True):
    @functools.partial(
        pl.kernel,
        out_type=(
            jax.ShapeDtypeStruct((_NP, 128), jnp.float32),
            jax.ShapeDtypeStruct((_NP, 128), jnp.float32),
        ),
        mesh=_mesh,
        scratch_types=[
            pltpu.VMEM((_RPW, 128), jnp.int32),
            pltpu.VMEM((_RPW, 128), jnp.int32),
            pltpu.VMEM((128, 128), jnp.float32),
            pltpu.VMEM((128, 128), jnp.float32),
            pltpu.VMEM_SHARED((_NP, 128), jnp.float32),
            pltpu.SemaphoreType.DMA,
            pltpu.SemaphoreType.DMA,
        ],
    )
    def k(th, srch, dsth, zh, out0, out1, src_v, dst_v, rows0, rows1,
          agg_sh, sem0, sem1):
        core = lax.axis_index("c")
        tid = lax.axis_index("s")

        def run(plane, out):
            wid = plane * 16 + tid
            for z in range(_ZR // 128):
                pltpu.sync_copy(
                    zh, agg_sh.at[pl.ds(tid * _ZR + z * 128, 128)])
            plsc.subcore_barrier()
            if gather:
                _pipe(th, srch, dsth, src_v, dst_v, rows0, rows1,
                      agg_sh, sem0, sem1, wid * _RPW, _RPW)
            else:
                pltpu.sync_copy(th.at[pl.ds(0, 128)], rows0)
                pltpu.sync_copy(dsth.at[pl.ds(wid * _RPW, _RPW)], dst_v)

                def body(j, carry):
                    pltpu.sync_copy(rows0, agg_sh.at[dst_v.at[j]], add=True)
                    return carry

                lax.fori_loop(0, _RPW, body, 0)
            plsc.subcore_barrier()
            pltpu.sync_copy(agg_sh.at[pl.ds(tid * _ZR, _ZR)],
                            out.at[pl.ds(tid * _ZR, _ZR)])

        @pl.when(core == 0)
        def _():
            run(0, out0)

        @pl.when(core == 1)
        def _():
            run(1, out1)

    return k(table, src2d, dst2d, zeros)


def _tc_first(x, W1, dp0, dp1):
    def body(x_ref, w_ref, d0_ref, d1_ref, u0_ref, u1_ref, di_ref):
        deg = d0_ref[:, 0:1] + d1_ref[:, 0:1] + 1.0
        dinv = 1.0 / jnp.sqrt(deg)
        u = jnp.dot(x_ref[...], w_ref[...],
                    preferred_element_type=jnp.float32)
        up = dinv * u
        u0_ref[...] = up[:, :128]
        u1_ref[...] = up[:, 128:]
        di_ref[...] = dinv

    return pl.pallas_call(
        body,
        grid=(_GRID,),
        in_specs=[
            pl.BlockSpec((_BN, _D), lambda i: (i, 0)),
            pl.BlockSpec((_D, _H), lambda i: (0, 0)),
            pl.BlockSpec((_BN, 128), lambda i: (i, 0)),
            pl.BlockSpec((_BN, 128), lambda i: (i, 0)),
        ],
        out_specs=[
            pl.BlockSpec((_BN, 128), lambda i: (i, 0)),
            pl.BlockSpec((_BN, 128), lambda i: (i, 0)),
            pl.BlockSpec((_BN, 1), lambda i: (i, 0)),
        ],
        out_shape=[
            jax.ShapeDtypeStruct((_N, 128), jnp.float32),
            jax.ShapeDtypeStruct((_N, 128), jnp.float32),
            jax.ShapeDtypeStruct((_N, 1), jnp.float32),
        ],
    )(x, W1, dp0, dp1)


def _tc_mid(agg0, agg1, u0, u1, dinv, W, gamma, delta, last):
    def body(a0_ref, a1_ref, u0_ref, u1_ref, di_ref, w_ref, g_ref, dl_ref,
             o1_ref, o2_ref):
        dinv_b = di_ref[...]
        s = jnp.concatenate(
            [a0_ref[...] + u0_ref[...], a1_ref[...] + u1_ref[...]], axis=1)
        h = jnp.maximum(g_ref[...] * (dinv_b * s) + dl_ref[...], 0.0)
        u = jnp.dot(h, w_ref[...], preferred_element_type=jnp.float32)
        up = dinv_b * u
        if last:
            o1_ref[...] = up
            o2_ref[...] = jnp.broadcast_to(up, (up.shape[0], 128))
        else:
            o1_ref[...] = up[:, :128]
            o2_ref[...] = up[:, 128:]

    wcols = 1 if last else _H
    out_specs = (
        [pl.BlockSpec((_BN, 1), lambda i: (i, 0)),
         pl.BlockSpec((_BN, 128), lambda i: (i, 0))]
        if last else
        [pl.BlockSpec((_BN, 128), lambda i: (i, 0)),
         pl.BlockSpec((_BN, 128), lambda i: (i, 0))]
    )
    out_shape = (
        [jax.ShapeDtypeStruct((_N, 1), jnp.float32),
         jax.ShapeDtypeStruct((_N, 128), jnp.float32)]
        if last else
        [jax.ShapeDtypeStruct((_N, 128), jnp.float32),
         jax.ShapeDtypeStruct((_N, 128), jnp.float32)]
    )
    return pl.pallas_call(
        body,
        grid=(_GRID,),
        in_specs=[
            pl.BlockSpec((_BN, 128), lambda i: (i, 0)),
            pl.BlockSpec((_BN, 128), lambda i: (i, 0)),
            pl.BlockSpec((_BN, 128), lambda i: (i, 0)),
            pl.BlockSpec((_BN, 128), lambda i: (i, 0)),
            pl.BlockSpec((_BN, 1), lambda i: (i, 0)),
            pl.BlockSpec((_H, wcols), lambda i: (0, 0)),
            pl.BlockSpec((1, _H), lambda i: (0, 0)),
            pl.BlockSpec((1, _H), lambda i: (0, 0)),
        ],
        out_specs=out_specs,
        out_shape=out_shape,
    )(agg0, agg1, u0, u1, dinv, W, gamma, delta)


def _tc_final(p0, p1, u4p, dinv, b4):
    def body(p0_ref, p1_ref, u_ref, di_ref, b_ref, o_ref):
        a = p0_ref[:, 0:1] + p1_ref[:, 0:1]
        o_ref[...] = di_ref[...] * (a + u_ref[...]) + b_ref[0, 0]

    return pl.pallas_call(
        body,
        grid=(_GRID,),
        in_specs=[
            pl.BlockSpec((_BN, 128), lambda i: (i, 0)),
            pl.BlockSpec((_BN, 128), lambda i: (i, 0)),
            pl.BlockSpec((_BN, 1), lambda i: (i, 0)),
            pl.BlockSpec((_BN, 1), lambda i: (i, 0)),
            pl.BlockSpec((1, 1), lambda i: (0, 0)),
        ],
        out_specs=pl.BlockSpec((_BN, 1), lambda i: (i, 0)),
        out_shape=jax.ShapeDtypeStruct((_N, 1), jnp.float32),
    )(p0, p1, u4p, dinv, b4)


def kernel(x, edge_index, W1, b1, g1, be1, m1, v1, W2, b2, g2, be2, m2, v2,
           W3, b3, g3, be3, m3, v3, W4, b4):
    E = edge_index.shape[1]
    pad = _EP - E
    ar = jnp.arange(pad, dtype=jnp.int32)
    src = jnp.concatenate([edge_index[0], (ar * 97) % _N])
    dst = jnp.concatenate([edge_index[1], _N + (ar % (_NP - _N))])
    src2d = src.reshape(_ROWS, 128)
    dst2d = dst.reshape(_ROWS, 128)
    zeros = jnp.zeros((128, 128), jnp.float32)
    ones = jnp.ones((128, 128), jnp.float32)

    eps = 1e-5
    g1a = g1 / jnp.sqrt(v1 + eps)
    g2a = g2 / jnp.sqrt(v2 + eps)
    g3a = g3 / jnp.sqrt(v3 + eps)
    d1 = (g1a * (b1 - m1) + be1).reshape(1, _H)
    d2 = (g2a * (b2 - m2) + be2).reshape(1, _H)
    d3 = (g3a * (b3 - m3) + be3).reshape(1, _H)
    g1a = g1a.reshape(1, _H)
    g2a = g2a.reshape(1, _H)
    g3a = g3a.reshape(1, _H)

    dp0, dp1 = _seg_sum_edge(ones, src2d, dst2d, zeros, gather=False)
    u0, u1, dinv = _tc_first(x, W1, dp0[:_N], dp1[:_N])

    a0, a1 = _seg_sum_wide(u0, u1, src2d, dst2d, zeros)
    u0, u1 = _tc_mid(a0[:_N], a1[:_N], u0, u1, dinv, W2, g1a, d1, last=False)

    a0, a1 = _seg_sum_wide(u0, u1, src2d, dst2d, zeros)
    u0, u1 = _tc_mid(a0[:_N], a1[:_N], u0, u1, dinv, W3, g2a, d2, last=False)

    a0, a1 = _seg_sum_wide(u0, u1, src2d, dst2d, zeros)
    u4p, u4p128 = _tc_mid(a0[:_N], a1[:_N], u0, u1, dinv, W4.reshape(_H, 1),
                          g3a, d3, last=True)

    p0, p1 = _seg_sum_edge(u4p128, src2d, dst2d, zeros)
    out = _tc_final(p0[:_N], p1[:_N], u4p, dinv, b4.reshape(1, 1))
    return out.reshape(-1)


# speedup vs baseline: 15.6149x; 1.0914x over previous
import functools

import jax
import jax.numpy as jnp
from jax import lax
from jax.experimental import pallas as pl
from jax.experimental.pallas import tpu as pltpu
from jax.experimental.pallas import tpu_sc as plsc

_N = 10000
_D = 256
_H = 256
_NP = 10240
_EP = 163840
_ROWS = _EP // 128
_RPT = _ROWS // 16
_RPW = _ROWS // 32
_ZR = _NP // 16
_BN = 1000
_GRID = _N // _BN

_mesh = plsc.VectorSubcoreMesh(core_axis_name="c", subcore_axis_name="s")


def _pipe(table, srch, dsth, src_v, dst_v, rows0, rows1, agg_sh, sem0, sem1,
          idx_base, nchunks):
    pltpu.sync_copy(srch.at[pl.ds(idx_base, nchunks)], src_v)
    pltpu.sync_copy(dsth.at[pl.ds(idx_base, nchunks)], dst_v)

    def start(j, buf, sem):
        pltpu.async_copy(table.at[src_v.at[j]], buf, sem)

    def wait(buf, sem):
        pltpu.make_async_copy(table.at[src_v.at[0]], buf, sem).wait()

    start(0, rows0, sem0)
    start(1, rows1, sem1)

    def body(j2, carry):
        b = 2 * j2
        wait(rows0, sem0)
        pltpu.sync_copy(rows0, agg_sh.at[dst_v.at[b]], add=True)
        start(b + 2, rows0, sem0)
        wait(rows1, sem1)
        pltpu.sync_copy(rows1, agg_sh.at[dst_v.at[b + 1]], add=True)
        start(b + 3, rows1, sem1)
        return carry

    lax.fori_loop(0, nchunks // 2 - 1, body, 0)
    wait(rows0, sem0)
    pltpu.sync_copy(rows0, agg_sh.at[dst_v.at[nchunks - 2]], add=True)
    wait(rows1, sem1)
    pltpu.sync_copy(rows1, agg_sh.at[dst_v.at[nchunks - 1]], add=True)


def _seg_sum_wide(u0, u1, src2d, dst2d, zeros):
    @functools.partial(
        pl.kernel,
        out_type=(
            jax.ShapeDtypeStruct((_NP, 128), jnp.float32),
            jax.ShapeDtypeStruct((_NP, 128), jnp.float32),
        ),
        mesh=_mesh,
        scratch_types=[
            pltpu.VMEM((_RPT // 2, 128), jnp.int32),
            pltpu.VMEM((_RPT // 2, 128), jnp.int32),
            pltpu.VMEM((128, 128), jnp.float32),
            pltpu.VMEM((128, 128), jnp.float32),
            pltpu.VMEM_SHARED((_NP, 128), jnp.float32),
            pltpu.SemaphoreType.DMA,
            pltpu.SemaphoreType.DMA,
        ],
    )
    def k(u0h, u1h, srch, dsth, zh, out0, out1, src_v, dst_v, rows0, rows1,
          agg_sh, sem0, sem1):
        core = lax.axis_index("c")
        tid = lax.axis_index("s")

        def run(table, out):
            for z in range(_ZR // 128):
                pltpu.sync_copy(
                    zh, agg_sh.at[pl.ds(tid * _ZR + z * 128, 128)])
            plsc.subcore_barrier()
            for phase in range(2):
                _pipe(table, srch, dsth, src_v, dst_v, rows0, rows1,
                      agg_sh, sem0, sem1,
                      tid * _RPT + phase * (_RPT // 2), _RPT // 2)
            plsc.subcore_barrier()
            pltpu.sync_copy(agg_sh.at[pl.ds(tid * _ZR, _ZR)],
                            out.at[pl.ds(tid * _ZR, _ZR)])

        @pl.when(core == 0)
        def _():
            run(u0h, out0)

        @pl.when(core == 1)
        def _():
            run(u1h, out1)

    return k(u0, u1, src2d, dst2d, zeros)


def _seg_sum_scalar(table, src2d, dst2d, zeros1, gather=True):
    @functools.partial(
        pl.kernel,
        out_type=jax.ShapeDtypeStruct((2, _NP), jnp.float32),
        mesh=_mesh,
        scratch_types=[
            pltpu.VMEM((_RPW, 128), jnp.int32),
            pltpu.VMEM((_RPW, 128), jnp.int32),
            pltpu.VMEM((128,), jnp.float32),
            pltpu.VMEM((128,), jnp.float32),
            pltpu.VMEM_SHARED((_NP,), jnp.float32),
            pltpu.SemaphoreType.DMA,
            pltpu.SemaphoreType.DMA,
        ],
    )
    def k(th, srch, dsth, zh, outh, src_v, dst_v, val0, val1, agg_sh,
          sem0, sem1):
        core = lax.axis_index("c")
        tid = lax.axis_index("s")

        def run(plane):
            wid = plane * 16 + tid
            pltpu.sync_copy(dsth.at[pl.ds(wid * _RPW, _RPW)], dst_v)
            for z in range(_ZR // 128):
                pltpu.sync_copy(
                    zh, agg_sh.at[pl.ds(tid * _ZR + z * 128, 128)])
            if gather:
                pltpu.sync_copy(srch.at[pl.ds(wid * _RPW, _RPW)], src_v)
            else:
                pltpu.sync_copy(th.at[pl.ds(0, 128)], val0)
            plsc.subcore_barrier()

            if gather:
                def start(j, buf, sem):
                    pltpu.async_copy(th.at[src_v.at[j]], buf, sem)

                def wait(buf, sem):
                    pltpu.make_async_copy(
                        th.at[src_v.at[0]], buf, sem).wait()

                start(0, val0, sem0)
                start(1, val1, sem1)

                def body(j2, carry):
                    b = 2 * j2
                    wait(val0, sem0)
                    pltpu.sync_copy(val0, agg_sh.at[dst_v.at[b]], add=True)
                    start(b + 2, val0, sem0)
                    wait(val1, sem1)
                    pltpu.sync_copy(val1, agg_sh.at[dst_v.at[b + 1]],
                                    add=True)
                    start(b + 3, val1, sem1)
                    return carry

                lax.fori_loop(0, _RPW // 2 - 1, body, 0)
                wait(val0, sem0)
                pltpu.sync_copy(val0, agg_sh.at[dst_v.at[_RPW - 2]],
                                add=True)
                wait(val1, sem1)
                pltpu.sync_copy(val1, agg_sh.at[dst_v.at[_RPW - 1]],
                                add=True)
            else:
                def body(j, carry):
                    pltpu.sync_copy(val0, agg_sh.at[dst_v.at[j]], add=True)
                    return carry

                lax.fori_loop(0, _RPW, body, 0)
            plsc.subcore_barrier()
            pltpu.sync_copy(agg_sh.at[pl.ds(tid * _ZR, _ZR)],
                            outh.at[plane].at[pl.ds(tid * _ZR, _ZR)])

        @pl.when(core == 0)
        def _():
            run(0)

        @pl.when(core == 1)
        def _():
            run(1)

    return k(table, src2d, dst2d, zeros1)


def _tc_first(x, W1, dp0, dp1):
    def body(x_ref, w_ref, d0_ref, d1_ref, u0_ref, u1_ref, di_ref):
        deg = d0_ref[...] + d1_ref[...] + 1.0
        dinv = 1.0 / jnp.sqrt(deg)
        u = jnp.dot(x_ref[...], w_ref[...],
                    preferred_element_type=jnp.float32)
        up = dinv * u
        u0_ref[...] = up[:, :128]
        u1_ref[...] = up[:, 128:]
        di_ref[...] = dinv

    return pl.pallas_call(
        body,
        grid=(_GRID,),
        in_specs=[
            pl.BlockSpec((_BN, _D), lambda i: (i, 0)),
            pl.BlockSpec((_D, _H), lambda i: (0, 0)),
            pl.BlockSpec((_BN, 1), lambda i: (i, 0)),
            pl.BlockSpec((_BN, 1), lambda i: (i, 0)),
        ],
        out_specs=[
            pl.BlockSpec((_BN, 128), lambda i: (i, 0)),
            pl.BlockSpec((_BN, 128), lambda i: (i, 0)),
            pl.BlockSpec((_BN, 1), lambda i: (i, 0)),
        ],
        out_shape=[
            jax.ShapeDtypeStruct((_N, 128), jnp.float32),
            jax.ShapeDtypeStruct((_N, 128), jnp.float32),
            jax.ShapeDtypeStruct((_N, 1), jnp.float32),
        ],
    )(x, W1, dp0, dp1)


def _tc_mid(agg0, agg1, u0, u1, dinv, W, gamma, delta, last):
    def body(a0_ref, a1_ref, u0_ref, u1_ref, di_ref, w_ref, g_ref, dl_ref,
             o1_ref, o2_ref=None):
        dinv_b = di_ref[...]
        s = jnp.concatenate(
            [a0_ref[...] + u0_ref[...], a1_ref[...] + u1_ref[...]], axis=1)
        h = jnp.maximum(g_ref[...] * (dinv_b * s) + dl_ref[...], 0.0)
        u = jnp.dot(h, w_ref[...], preferred_element_type=jnp.float32)
        up = dinv_b * u
        if last:
            o1_ref[...] = up
        else:
            o1_ref[...] = up[:, :128]
            o2_ref[...] = up[:, 128:]

    wcols = 1 if last else _H
    out_specs = (
        [pl.BlockSpec((_BN, 1), lambda i: (i, 0))]
        if last else
        [pl.BlockSpec((_BN, 128), lambda i: (i, 0)),
         pl.BlockSpec((_BN, 128), lambda i: (i, 0))]
    )
    out_shape = (
        [jax.ShapeDtypeStruct((_N, 1), jnp.float32)]
        if last else
        [jax.ShapeDtypeStruct((_N, 128), jnp.float32),
         jax.ShapeDtypeStruct((_N, 128), jnp.float32)]
    )
    return pl.pallas_call(
        body,
        grid=(_GRID,),
        in_specs=[
            pl.BlockSpec((_BN, 128), lambda i: (i, 0)),
            pl.BlockSpec((_BN, 128), lambda i: (i, 0)),
            pl.BlockSpec((_BN, 128), lambda i: (i, 0)),
            pl.BlockSpec((_BN, 128), lambda i: (i, 0)),
            pl.BlockSpec((_BN, 1), lambda i: (i, 0)),
            pl.BlockSpec((_H, wcols), lambda i: (0, 0)),
            pl.BlockSpec((1, _H), lambda i: (0, 0)),
            pl.BlockSpec((1, _H), lambda i: (0, 0)),
        ],
        out_specs=out_specs,
        out_shape=out_shape,
    )(agg0, agg1, u0, u1, dinv, W, gamma, delta)


def _tc_final(p0, p1, u4p, dinv, b4):
    def body(p0_ref, p1_ref, u_ref, di_ref, b_ref, o_ref):
        a = p0_ref[...] + p1_ref[...]
        o_ref[...] = di_ref[...] * (a + u_ref[...]) + b_ref[0, 0]

    return pl.pallas_call(
        body,
        grid=(_GRID,),
        in_specs=[
            pl.BlockSpec((_BN, 1), lambda i: (i, 0)),
            pl.BlockSpec((_BN, 1), lambda i: (i, 0)),
            pl.BlockSpec((_BN, 1), lambda i: (i, 0)),
            pl.BlockSpec((_BN, 1), lambda i: (i, 0)),
            pl.BlockSpec((1, 1), lambda i: (0, 0)),
        ],
        out_specs=pl.BlockSpec((_BN, 1), lambda i: (i, 0)),
        out_shape=jax.ShapeDtypeStruct((_N, 1), jnp.float32),
    )(p0, p1, u4p, dinv, b4)


def kernel(x, edge_index, W1, b1, g1, be1, m1, v1, W2, b2, g2, be2, m2, v2,
           W3, b3, g3, be3, m3, v3, W4, b4):
    E = edge_index.shape[1]
    pad = _EP - E
    ar = jnp.arange(pad, dtype=jnp.int32)
    src = jnp.concatenate([edge_index[0], (ar * 97) % _N])
    dst = jnp.concatenate([edge_index[1], _N + (ar % (_NP - _N))])
    src2d = src.reshape(_ROWS, 128)
    dst2d = dst.reshape(_ROWS, 128)
    zeros = jnp.zeros((128, 128), jnp.float32)
    zeros1 = jnp.zeros((128,), jnp.float32)
    ones1 = jnp.ones((128,), jnp.float32)

    eps = 1e-5
    g1a = g1 / jnp.sqrt(v1 + eps)
    g2a = g2 / jnp.sqrt(v2 + eps)
    g3a = g3 / jnp.sqrt(v3 + eps)
    d1 = (g1a * (b1 - m1) + be1).reshape(1, _H)
    d2 = (g2a * (b2 - m2) + be2).reshape(1, _H)
    d3 = (g3a * (b3 - m3) + be3).reshape(1, _H)
    g1a = g1a.reshape(1, _H)
    g2a = g2a.reshape(1, _H)
    g3a = g3a.reshape(1, _H)

    degp = _seg_sum_scalar(ones1, src2d, dst2d, zeros1, gather=False)
    u0, u1, dinv = _tc_first(x, W1, degp[0, :_N].reshape(_N, 1),
                             degp[1, :_N].reshape(_N, 1))

    a0, a1 = _seg_sum_wide(u0, u1, src2d, dst2d, zeros)
    u0, u1 = _tc_mid(a0[:_N], a1[:_N], u0, u1, dinv, W2, g1a, d1, last=False)

    a0, a1 = _seg_sum_wide(u0, u1, src2d, dst2d, zeros)
    u0, u1 = _tc_mid(a0[:_N], a1[:_N], u0, u1, dinv, W3, g2a, d2, last=False)

    a0, a1 = _seg_sum_wide(u0, u1, src2d, dst2d, zeros)
    (u4p,) = _tc_mid(a0[:_N], a1[:_N], u0, u1, dinv, W4.reshape(_H, 1),
                     g3a, d3, last=True)

    aggp = _seg_sum_scalar(u4p.reshape(-1), src2d, dst2d, zeros1)
    out = _tc_final(aggp[0, :_N].reshape(_N, 1), aggp[1, :_N].reshape(_N, 1),
                    u4p, dinv, b4.reshape(1, 1))
    return out.reshape(-1)


# speedup vs baseline: 17.0730x; 1.0934x over previous
import functools

import jax
import jax.numpy as jnp
from jax import lax
from jax.experimental import pallas as pl
from jax.experimental.pallas import tpu as pltpu
from jax.experimental.pallas import tpu_sc as plsc

_N = 10000
_D = 256
_H = 256
_NP = 10240
_EP = 163840
_ROWS = _EP // 128
_RPT = _ROWS // 16
_RPW = _ROWS // 32
_ZR = _NP // 16
_BN = 1000
_GRID = _N // _BN

_mesh = plsc.VectorSubcoreMesh(core_axis_name="c", subcore_axis_name="s")


def _pipe(table, srch, dsth, src_v, dst_v, rows0, rows1, agg_sh, sem0, sem1,
          idx_base, nchunks):
    pltpu.sync_copy(srch.at[pl.ds(idx_base, nchunks)], src_v)
    pltpu.sync_copy(dsth.at[pl.ds(idx_base, nchunks)], dst_v)

    def start(j, buf, sem):
        pltpu.async_copy(table.at[src_v.at[j]], buf, sem)

    def wait(buf, sem):
        pltpu.make_async_copy(table.at[src_v.at[0]], buf, sem).wait()

    start(0, rows0, sem0)
    start(1, rows1, sem1)

    def body(j2, carry):
        b = 2 * j2
        wait(rows0, sem0)
        pltpu.sync_copy(rows0, agg_sh.at[dst_v.at[b]], add=True)
        start(b + 2, rows0, sem0)
        wait(rows1, sem1)
        pltpu.sync_copy(rows1, agg_sh.at[dst_v.at[b + 1]], add=True)
        start(b + 3, rows1, sem1)
        return carry

    lax.fori_loop(0, nchunks // 2 - 1, body, 0)
    wait(rows0, sem0)
    pltpu.sync_copy(rows0, agg_sh.at[dst_v.at[nchunks - 2]], add=True)
    wait(rows1, sem1)
    pltpu.sync_copy(rows1, agg_sh.at[dst_v.at[nchunks - 1]], add=True)


def _seg_sum_wide(u0, u1, src2d, dst2d):
    @functools.partial(
        pl.kernel,
        out_type=(
            jax.ShapeDtypeStruct((_NP, 128), jnp.float32),
            jax.ShapeDtypeStruct((_NP, 128), jnp.float32),
        ),
        mesh=_mesh,
        scratch_types=[
            pltpu.VMEM((_RPT // 2, 128), jnp.int32),
            pltpu.VMEM((_RPT // 2, 128), jnp.int32),
            pltpu.VMEM((128, 128), jnp.float32),
            pltpu.VMEM((128, 128), jnp.float32),
            pltpu.VMEM_SHARED((_NP, 128), jnp.float32),
            pltpu.SemaphoreType.DMA,
            pltpu.SemaphoreType.DMA,
        ],
    )
    def k(u0h, u1h, srch, dsth, out0, out1, src_v, dst_v, rows0, rows1,
          agg_sh, sem0, sem1):
        core = lax.axis_index("c")
        tid = lax.axis_index("s")

        def run(table, out):
            @pl.when(tid < 15)
            def _():
                pltpu.sync_copy(table.at[pl.ds(tid * _ZR, _ZR)],
                                agg_sh.at[pl.ds(tid * _ZR, _ZR)])

            @pl.when(tid == 15)
            def _():
                pltpu.sync_copy(table.at[pl.ds(15 * _ZR, _N - 15 * _ZR)],
                                agg_sh.at[pl.ds(15 * _ZR, _N - 15 * _ZR)])

            plsc.subcore_barrier()
            for phase in range(2):
                _pipe(table, srch, dsth, src_v, dst_v, rows0, rows1,
                      agg_sh, sem0, sem1,
                      tid * _RPT + phase * (_RPT // 2), _RPT // 2)
            plsc.subcore_barrier()
            pltpu.sync_copy(agg_sh.at[pl.ds(tid * _ZR, _ZR)],
                            out.at[pl.ds(tid * _ZR, _ZR)])

        @pl.when(core == 0)
        def _():
            run(u0h, out0)

        @pl.when(core == 1)
        def _():
            run(u1h, out1)

    return k(u0, u1, src2d, dst2d)


def _seg_sum_scalar(table, table2, src2d, dst2d, zeros1, gather=True):
    @functools.partial(
        pl.kernel,
        out_type=jax.ShapeDtypeStruct((2, _NP), jnp.float32),
        mesh=_mesh,
        scratch_types=[
            pltpu.VMEM((_RPW, 128), jnp.int32),
            pltpu.VMEM((_RPW, 128), jnp.int32),
            pltpu.VMEM((128,), jnp.float32),
            pltpu.VMEM((128,), jnp.float32),
            pltpu.VMEM((8, 128), jnp.float32),
            pltpu.VMEM_SHARED((_NP,), jnp.float32),
            pltpu.SemaphoreType.DMA,
            pltpu.SemaphoreType.DMA,
        ],
    )
    def k(th, t2h, srch, dsth, zh, outh, src_v, dst_v, val0, val1, ini_v,
          agg_sh, sem0, sem1):
        core = lax.axis_index("c")
        tid = lax.axis_index("s")

        def run(plane):
            wid = plane * 16 + tid
            pltpu.sync_copy(dsth.at[pl.ds(wid * _RPW, _RPW)], dst_v)
            if gather and plane == 0:
                @pl.when(tid < 10)
                def _():
                    pltpu.sync_copy(t2h.at[pl.ds(tid * 8, 8)], ini_v)
                    for z in range(8):
                        pltpu.sync_copy(
                            ini_v.at[z],
                            agg_sh.at[pl.ds(tid * 1024 + z * 128, 128)])
            else:
                for z in range(_ZR // 128):
                    pltpu.sync_copy(
                        zh, agg_sh.at[pl.ds(tid * _ZR + z * 128, 128)])
            if gather:
                pltpu.sync_copy(srch.at[pl.ds(wid * _RPW, _RPW)], src_v)
            else:
                pltpu.sync_copy(th.at[pl.ds(0, 128)], val0)
            plsc.subcore_barrier()

            if gather:
                def start(j, buf, sem):
                    pltpu.async_copy(th.at[src_v.at[j]], buf, sem)

                def wait(buf, sem):
                    pltpu.make_async_copy(
                        th.at[src_v.at[0]], buf, sem).wait()

                start(0, val0, sem0)
                start(1, val1, sem1)

                def body(j2, carry):
                    b = 2 * j2
                    wait(val0, sem0)
                    pltpu.sync_copy(val0, agg_sh.at[dst_v.at[b]], add=True)
                    start(b + 2, val0, sem0)
                    wait(val1, sem1)
                    pltpu.sync_copy(val1, agg_sh.at[dst_v.at[b + 1]],
                                    add=True)
                    start(b + 3, val1, sem1)
                    return carry

                lax.fori_loop(0, _RPW // 2 - 1, body, 0)
                wait(val0, sem0)
                pltpu.sync_copy(val0, agg_sh.at[dst_v.at[_RPW - 2]],
                                add=True)
                wait(val1, sem1)
                pltpu.sync_copy(val1, agg_sh.at[dst_v.at[_RPW - 1]],
                                add=True)
            else:
                def body(j, carry):
                    pltpu.sync_copy(val0, agg_sh.at[dst_v.at[j]], add=True)
                    return carry

                lax.fori_loop(0, _RPW, body, 0)
            plsc.subcore_barrier()
            pltpu.sync_copy(agg_sh.at[pl.ds(tid * _ZR, _ZR)],
                            outh.at[plane].at[pl.ds(tid * _ZR, _ZR)])

        @pl.when(core == 0)
        def _():
            run(0)

        @pl.when(core == 1)
        def _():
            run(1)

    return k(table, table2, src2d, dst2d, zeros1)


def _tc_first(x, W1, dp0, dp1):
    def body(x_ref, w_ref, d0_ref, d1_ref, u0_ref, u1_ref, di_ref):
        deg = d0_ref[...] + d1_ref[...] + 1.0
        dinv = 1.0 / jnp.sqrt(deg)
        u = jnp.dot(x_ref[...], w_ref[...],
                    preferred_element_type=jnp.float32)
        up = dinv * u
        u0_ref[...] = up[:, :128]
        u1_ref[...] = up[:, 128:]
        di_ref[...] = dinv

    return pl.pallas_call(
        body,
        grid=(_GRID,),
        in_specs=[
            pl.BlockSpec((_BN, _D), lambda i: (i, 0)),
            pl.BlockSpec((_D, _H), lambda i: (0, 0)),
            pl.BlockSpec((_BN, 1), lambda i: (i, 0)),
            pl.BlockSpec((_BN, 1), lambda i: (i, 0)),
        ],
        out_specs=[
            pl.BlockSpec((_BN, 128), lambda i: (i, 0)),
            pl.BlockSpec((_BN, 128), lambda i: (i, 0)),
            pl.BlockSpec((_BN, 1), lambda i: (i, 0)),
        ],
        out_shape=[
            jax.ShapeDtypeStruct((_N, 128), jnp.float32),
            jax.ShapeDtypeStruct((_N, 128), jnp.float32),
            jax.ShapeDtypeStruct((_N, 1), jnp.float32),
        ],
    )(x, W1, dp0, dp1)


def _tc_mid(agg0, agg1, dinv, W, gamma, delta, last):
    def body(a0_ref, a1_ref, di_ref, w_ref, g_ref, dl_ref,
             o1_ref, o2_ref=None):
        dinv_b = di_ref[...]
        s = jnp.concatenate([a0_ref[...], a1_ref[...]], axis=1)
        h = jnp.maximum(g_ref[...] * (dinv_b * s) + dl_ref[...], 0.0)
        u = jnp.dot(h, w_ref[...], preferred_element_type=jnp.float32)
        up = dinv_b * u
        if last:
            o1_ref[...] = up
        else:
            o1_ref[...] = up[:, :128]
            o2_ref[...] = up[:, 128:]

    wcols = 1 if last else _H
    out_specs = (
        [pl.BlockSpec((_BN, 1), lambda i: (i, 0))]
        if last else
        [pl.BlockSpec((_BN, 128), lambda i: (i, 0)),
         pl.BlockSpec((_BN, 128), lambda i: (i, 0))]
    )
    out_shape = (
        [jax.ShapeDtypeStruct((_N, 1), jnp.float32)]
        if last else
        [jax.ShapeDtypeStruct((_N, 128), jnp.float32),
         jax.ShapeDtypeStruct((_N, 128), jnp.float32)]
    )
    return pl.pallas_call(
        body,
        grid=(_GRID,),
        in_specs=[
            pl.BlockSpec((_BN, 128), lambda i: (i, 0)),
            pl.BlockSpec((_BN, 128), lambda i: (i, 0)),
            pl.BlockSpec((_BN, 1), lambda i: (i, 0)),
            pl.BlockSpec((_H, wcols), lambda i: (0, 0)),
            pl.BlockSpec((1, _H), lambda i: (0, 0)),
            pl.BlockSpec((1, _H), lambda i: (0, 0)),
        ],
        out_specs=out_specs,
        out_shape=out_shape,
    )(agg0, agg1, dinv, W, gamma, delta)


def _tc_final(p0, p1, dinv, b4):
    def body(p0_ref, p1_ref, di_ref, b_ref, o_ref):
        o_ref[...] = di_ref[...] * (p0_ref[...] + p1_ref[...]) + b_ref[0, 0]

    return pl.pallas_call(
        body,
        grid=(_GRID,),
        in_specs=[
            pl.BlockSpec((_BN, 1), lambda i: (i, 0)),
            pl.BlockSpec((_BN, 1), lambda i: (i, 0)),
            pl.BlockSpec((_BN, 1), lambda i: (i, 0)),
            pl.BlockSpec((1, 1), lambda i: (0, 0)),
        ],
        out_specs=pl.BlockSpec((_BN, 1), lambda i: (i, 0)),
        out_shape=jax.ShapeDtypeStruct((_N, 1), jnp.float32),
    )(p0, p1, dinv, b4)


def kernel(x, edge_index, W1, b1, g1, be1, m1, v1, W2, b2, g2, be2, m2, v2,
           W3, b3, g3, be3, m3, v3, W4, b4):
    E = edge_index.shape[1]
    pad = _EP - E
    ar = jnp.arange(pad, dtype=jnp.int32)
    src = jnp.concatenate([edge_index[0], (ar * 97) % _N])
    dst = jnp.concatenate([edge_index[1], _N + (ar % (_NP - _N))])
    src2d = src.reshape(_ROWS, 128)
    dst2d = dst.reshape(_ROWS, 128)
    zeros1 = jnp.zeros((128,), jnp.float32)
    ones1 = jnp.ones((128,), jnp.float32)

    eps = 1e-5
    g1a = g1 / jnp.sqrt(v1 + eps)
    g2a = g2 / jnp.sqrt(v2 + eps)
    g3a = g3 / jnp.sqrt(v3 + eps)
    d1 = (g1a * (b1 - m1) + be1).reshape(1, _H)
    d2 = (g2a * (b2 - m2) + be2).reshape(1, _H)
    d3 = (g3a * (b3 - m3) + be3).reshape(1, _H)
    g1a = g1a.reshape(1, _H)
    g2a = g2a.reshape(1, _H)
    g3a = g3a.reshape(1, _H)

    degp = _seg_sum_scalar(ones1, jnp.zeros((_NP // 128, 128), jnp.float32),
                           src2d, dst2d, zeros1, gather=False)
    u0, u1, dinv = _tc_first(x, W1, degp[0, :_N].reshape(_N, 1),
                             degp[1, :_N].reshape(_N, 1))

    a0, a1 = _seg_sum_wide(u0, u1, src2d, dst2d)
    u0, u1 = _tc_mid(a0[:_N], a1[:_N], dinv, W2, g1a, d1, last=False)

    a0, a1 = _seg_sum_wide(u0, u1, src2d, dst2d)
    u0, u1 = _tc_mid(a0[:_N], a1[:_N], dinv, W3, g2a, d2, last=False)

    a0, a1 = _seg_sum_wide(u0, u1, src2d, dst2d)
    (u4p,) = _tc_mid(a0[:_N], a1[:_N], dinv, W4.reshape(_H, 1),
                     g3a, d3, last=True)

    u4pp = jnp.concatenate(
        [u4p.reshape(-1), jnp.zeros((_NP - _N,), jnp.float32)])
    aggp = _seg_sum_scalar(u4pp, u4pp.reshape(_NP // 128, 128),
                           src2d, dst2d, zeros1)
    out = _tc_final(aggp[0, :_N].reshape(_N, 1), aggp[1, :_N].reshape(_N, 1),
                    dinv, b4.reshape(1, 1))
    return out.reshape(-1)
